# baseline (speedup 1.0000x reference)
import jax
import jax.numpy as jnp
from jax import lax
from jax.experimental import pallas as pl
from jax.experimental.pallas import tpu as pltpu
from jax.experimental.pallas import tpu_sc as plsc

NUM_ENTITIES = 1000000
NUM_RELATIONS = 1000
HALF_DIM = 64
ROW = 2 * HALF_DIM
BATCH = 16384

_NC = 2
_NS = 16
_NW = _NC * _NS
_PER_W = BATCH // _NW
_CHUNK = 32
_NCHUNK = _PER_W // _CHUNK


def _trig_body(rt_ref, trig_ref):
    r = rt_ref[...].T
    trig_ref[...] = jnp.concatenate([jnp.cos(r), jnp.sin(r)], axis=-1)


def _trig_tables(relation_emb_t):
    return pl.pallas_call(
        _trig_body,
        out_shape=jax.ShapeDtypeStruct((NUM_RELATIONS, ROW), jnp.float32),
    )(relation_emb_t)


def _newton_sqrt(x):
    i = jnp.int32(0x5F3759DF) - (lax.bitcast_convert_type(x, jnp.int32) >> 1)
    y = lax.bitcast_convert_type(i, jnp.float32)
    half = jnp.float32(0.5) * x
    y = y * (jnp.float32(1.5) - half * y * y)
    return x * y


def _sc_body(heads_hbm, rels_hbm, tails_hbm, entity_hbm, trig_hbm,
             out_hbm, hidx_v, ridx_v, tidx_v, h_v, t_v, trig_v, out_v,
             gsem0, gsem1, osem0, osem1):
    wid = lax.axis_index("s") * _NC + lax.axis_index("c")
    base_w = wid * _PER_W
    wsl = pl.ds(base_w, _PER_W)
    cp_hi = pltpu.async_copy(heads_hbm.at[wsl], hidx_v, osem0)
    cp_ri = pltpu.async_copy(rels_hbm.at[wsl], ridx_v, osem0)
    cp_ti = pltpu.async_copy(tails_hbm.at[wsl], tidx_v, osem0)
    cp_hi.wait()
    cp_ri.wait()
    cp_ti.wait()

    gsems = (gsem0, gsem1)
    osems = (osem0, osem1)

    def issue(cdyn, b):
        csl = pl.ds(cdyn * _CHUNK, _CHUNK)
        return (
            pltpu.async_copy(entity_hbm.at[hidx_v.at[csl]], h_v.at[b], gsems[b]),
            pltpu.async_copy(entity_hbm.at[tidx_v.at[csl]], t_v.at[b], gsems[b]),
            pltpu.async_copy(trig_hbm.at[ridx_v.at[csl]], trig_v.at[b], gsems[b]),
        )

    def drain_gather(b):
        pltpu.make_async_copy(entity_hbm.at[pl.ds(0, _CHUNK)], h_v.at[b],
                              gsems[b]).wait()
        pltpu.make_async_copy(entity_hbm.at[pl.ds(0, _CHUNK)], t_v.at[b],
                              gsems[b]).wait()
        pltpu.make_async_copy(trig_hbm.at[pl.ds(0, _CHUNK)], trig_v.at[b],
                              gsems[b]).wait()

    def drain_out(b):
        pltpu.make_async_copy(out_v.at[b], out_hbm.at[pl.ds(0, _CHUNK)],
                              osems[b]).wait()

    lane = lax.iota(jnp.int32, 16)
    issue(0, 0)
    issue(1, 1)

    def half_body(p, b):
        c = 2 * p + b
        drain_gather(b)

        @pl.when(p > 0)
        def _():
            drain_out(b)

        def group_body(gi, carry):
            score_vec = jnp.zeros((16,), jnp.float32)
            for k in range(16):
                i = gi * 16 + k
                acc = jnp.zeros((16,), jnp.float32)
                for g in range(HALF_DIM // 16):
                    re_sl = pl.ds(g * 16, 16)
                    im_sl = pl.ds(HALF_DIM + g * 16, 16)
                    hre = h_v[b, i, re_sl]
                    him = h_v[b, i, im_sl]
                    tre = t_v[b, i, re_sl]
                    tim = t_v[b, i, im_sl]
                    co = trig_v[b, i, re_sl]
                    si = trig_v[b, i, im_sl]
                    hr_re = hre * co - him * si
                    hr_im = hre * si + him * co
                    dre = hr_re - tre
                    dim_ = hr_im - tim
                    acc = acc + _newton_sqrt(dre * dre + dim_ * dim_)
                score_vec = jnp.where(lane == k, jnp.sum(acc), score_vec)
            out_v[b, pl.ds(gi * 16, 16)] = score_vec
            return carry

        lax.fori_loop(0, _CHUNK // 16, group_body, 0)
        pltpu.async_copy(
            out_v.at[b], out_hbm.at[pl.ds(base_w + c * _CHUNK, _CHUNK)],
            osems[b])

        @pl.when(p + 1 < _NCHUNK // 2)
        def _():
            issue(c + 2, b)

    def pair_body(p, carry):
        half_body(p, 0)
        half_body(p, 1)
        return carry

    lax.fori_loop(0, _NCHUNK // 2, pair_body, 0)
    drain_out(0)
    drain_out(1)


@jax.jit
def _rotate_score(heads, rels, tails, entity_emb, trig_t):
    mesh = plsc.VectorSubcoreMesh(core_axis_name="c", subcore_axis_name="s")
    kfn = pl.kernel(
        _sc_body,
        out_type=jax.ShapeDtypeStruct((BATCH,), jnp.float32),
        mesh=mesh,
        compiler_params=pltpu.CompilerParams(needs_layout_passes=False),
        scratch_types=[
            pltpu.VMEM((_PER_W,), jnp.int32),
            pltpu.VMEM((_PER_W,), jnp.int32),
            pltpu.VMEM((_PER_W,), jnp.int32),
            pltpu.VMEM((2, _CHUNK, ROW), jnp.float32),
            pltpu.VMEM((2, _CHUNK, ROW), jnp.float32),
            pltpu.VMEM((2, _CHUNK, ROW), jnp.float32),
            pltpu.VMEM((2, _CHUNK), jnp.float32),
            pltpu.SemaphoreType.DMA,
            pltpu.SemaphoreType.DMA,
            pltpu.SemaphoreType.DMA,
            pltpu.SemaphoreType.DMA,
        ],
    )
    return kfn(heads, rels, tails, entity_emb, trig_t)


def kernel(heads, relations, tails, entity_emb, relation_emb):
    heads = heads.astype(jnp.int32)
    relations = relations.astype(jnp.int32)
    tails = tails.astype(jnp.int32)
    trig_t = _trig_tables(relation_emb.T)
    return _rotate_score(heads, relations, tails, entity_emb, trig_t)


# speedup vs baseline: 1.0398x; 1.0398x over previous
import jax
import jax.numpy as jnp
from jax import lax
from jax.experimental import pallas as pl
from jax.experimental.pallas import tpu as pltpu
from jax.experimental.pallas import tpu_sc as plsc

NUM_ENTITIES = 1000000
NUM_RELATIONS = 1000
HALF_DIM = 64
ROW = 2 * HALF_DIM
BATCH = 16384

_NC = 2
_NS = 16
_NW = _NC * _NS
_PER_W = BATCH // _NW
_CHUNK = 64
_NCHUNK = _PER_W // _CHUNK


def _trig_body(rt_ref, trig_ref):
    r = rt_ref[...].T
    trig_ref[...] = jnp.concatenate([jnp.cos(r), jnp.sin(r)], axis=-1)


def _trig_tables(relation_emb_t):
    return pl.pallas_call(
        _trig_body,
        out_shape=jax.ShapeDtypeStruct((NUM_RELATIONS, ROW), jnp.float32),
    )(relation_emb_t)


def _newton_sqrt(x):
    i = jnp.int32(0x5F3759DF) - (lax.bitcast_convert_type(x, jnp.int32) >> 1)
    y = lax.bitcast_convert_type(i, jnp.float32)
    half = jnp.float32(0.5) * x
    y = y * (jnp.float32(1.5) - half * y * y)
    return x * y


def _sc_body(heads_hbm, rels_hbm, tails_hbm, entity_hbm, trig_hbm,
             out_hbm, hidx_v, ridx_v, tidx_v, h_v, t_v, trig_v, out_v,
             gsem0, gsem1, osem0, osem1):
    wid = lax.axis_index("s") * _NC + lax.axis_index("c")
    base_w = wid * _PER_W
    wsl = pl.ds(base_w, _PER_W)
    cp_hi = pltpu.async_copy(heads_hbm.at[wsl], hidx_v, osem0)
    cp_ri = pltpu.async_copy(rels_hbm.at[wsl], ridx_v, osem0)
    cp_ti = pltpu.async_copy(tails_hbm.at[wsl], tidx_v, osem0)
    cp_hi.wait()
    cp_ri.wait()
    cp_ti.wait()

    gsems = (gsem0, gsem1)
    osems = (osem0, osem1)

    def issue(cdyn, b):
        csl = pl.ds(cdyn * _CHUNK, _CHUNK)
        return (
            pltpu.async_copy(entity_hbm.at[hidx_v.at[csl]], h_v.at[b], gsems[b]),
            pltpu.async_copy(entity_hbm.at[tidx_v.at[csl]], t_v.at[b], gsems[b]),
            pltpu.async_copy(trig_hbm.at[ridx_v.at[csl]], trig_v.at[b], gsems[b]),
        )

    def drain_gather(b):
        pltpu.make_async_copy(entity_hbm.at[pl.ds(0, _CHUNK)], h_v.at[b],
                              gsems[b]).wait()
        pltpu.make_async_copy(entity_hbm.at[pl.ds(0, _CHUNK)], t_v.at[b],
                              gsems[b]).wait()
        pltpu.make_async_copy(trig_hbm.at[pl.ds(0, _CHUNK)], trig_v.at[b],
                              gsems[b]).wait()

    def drain_out(b):
        pltpu.make_async_copy(out_v.at[b], out_hbm.at[pl.ds(0, _CHUNK)],
                              osems[b]).wait()

    lane = lax.iota(jnp.int32, 16)
    issue(0, 0)
    issue(1, 1)

    def half_body(p, b):
        c = 2 * p + b
        drain_gather(b)

        @pl.when(p > 0)
        def _():
            drain_out(b)

        def group_body(gi, carry):
            score_vec = jnp.zeros((16,), jnp.float32)
            for k in range(16):
                i = gi * 16 + k
                acc = jnp.zeros((16,), jnp.float32)
                for g in range(HALF_DIM // 16):
                    re_sl = pl.ds(g * 16, 16)
                    im_sl = pl.ds(HALF_DIM + g * 16, 16)
                    hre = h_v[b, i, re_sl]
                    him = h_v[b, i, im_sl]
                    tre = t_v[b, i, re_sl]
                    tim = t_v[b, i, im_sl]
                    co = trig_v[b, i, re_sl]
                    si = trig_v[b, i, im_sl]
                    hr_re = hre * co - him * si
                    hr_im = hre * si + him * co
                    dre = hr_re - tre
                    dim_ = hr_im - tim
                    acc = acc + _newton_sqrt(dre * dre + dim_ * dim_)
                score_vec = jnp.where(lane == k, jnp.sum(acc), score_vec)
            out_v[b, pl.ds(gi * 16, 16)] = score_vec
            return carry

        lax.fori_loop(0, _CHUNK // 16, group_body, 0)
        pltpu.async_copy(
            out_v.at[b], out_hbm.at[pl.ds(base_w + c * _CHUNK, _CHUNK)],
            osems[b])

        @pl.when(p + 1 < _NCHUNK // 2)
        def _():
            issue(c + 2, b)

    def pair_body(p, carry):
        half_body(p, 0)
        half_body(p, 1)
        return carry

    lax.fori_loop(0, _NCHUNK // 2, pair_body, 0)
    drain_out(0)
    drain_out(1)


@jax.jit
def _rotate_score(heads, rels, tails, entity_emb, trig_t):
    mesh = plsc.VectorSubcoreMesh(core_axis_name="c", subcore_axis_name="s")
    kfn = pl.kernel(
        _sc_body,
        out_type=jax.ShapeDtypeStruct((BATCH,), jnp.float32),
        mesh=mesh,
        compiler_params=pltpu.CompilerParams(needs_layout_passes=False),
        scratch_types=[
            pltpu.VMEM((_PER_W,), jnp.int32),
            pltpu.VMEM((_PER_W,), jnp.int32),
            pltpu.VMEM((_PER_W,), jnp.int32),
            pltpu.VMEM((2, _CHUNK, ROW), jnp.float32),
            pltpu.VMEM((2, _CHUNK, ROW), jnp.float32),
            pltpu.VMEM((2, _CHUNK, ROW), jnp.float32),
            pltpu.VMEM((2, _CHUNK), jnp.float32),
            pltpu.SemaphoreType.DMA,
            pltpu.SemaphoreType.DMA,
            pltpu.SemaphoreType.DMA,
            pltpu.SemaphoreType.DMA,
        ],
    )
    return kfn(heads, rels, tails, entity_emb, trig_t)


def kernel(heads, relations, tails, entity_emb, relation_emb):
    heads = heads.astype(jnp.int32)
    relations = relations.astype(jnp.int32)
    tails = tails.astype(jnp.int32)
    trig_t = _trig_tables(relation_emb.T)
    return _rotate_score(heads, relations, tails, entity_emb, trig_t)
